# baseline (speedup 1.0000x reference)
import functools

import jax
import jax.numpy as jnp
from jax import lax
from jax.experimental import pallas as pl
from jax.experimental.pallas import tpu as pltpu
from jax.experimental.pallas import tpu_sc as plsc

N = 10000
E = 320000
D = 128
H = 16

NC = 2
NS = 16
NW = NC * NS
CH = 128
EPW = -(-E // (NW * CH * 4)) * CH * 4
NJ = EPW // CH
EPAD = NW * EPW
RPT = 640
NP = NS * RPT
NBUF = 4
NJB = NJ // NBUF

_mesh = plsc.VectorSubcoreMesh(core_axis_name="core", subcore_axis_name="subcore")
_sc_params = pltpu.CompilerParams(needs_layout_passes=False,
                                  use_tc_tiling_on_sc=False)


def _scale_rows(gbuf, sbuf, w_v, j, b):
    for q in range(CH // 16):
        wv = w_v[j, b, pl.ds(q * 16, 16)]
        ridx = lax.iota(jnp.int32, 16) + q * 16
        for f in range(H):
            fidx = jnp.full((16,), f, dtype=jnp.int32)
            vals = plsc.load_gather(gbuf, [ridx, fidx])
            plsc.store_scatter(sbuf, [ridx, fidx], vals * wv)


def _zero_shared_stripe(zbuf_v, acc_sh, s):
    @pl.loop(0, RPT)
    def _(i):
        zbuf_v[i, :] = jnp.zeros((16,), jnp.float32)
    pltpu.sync_copy(zbuf_v, acc_sh.at[pl.ds(s * RPT, RPT)])


def _sc_degree(dst3, w3):
    @functools.partial(
        pl.kernel,
        out_type=jax.ShapeDtypeStruct((NC, NP, H), jnp.float32),
        mesh=_mesh,
        compiler_params=_sc_params,
        scratch_types=[
            pltpu.VMEM((NJB, NBUF, CH), jnp.int32),
            pltpu.VMEM((NJB, NBUF, CH), jnp.float32),
            pltpu.VMEM((CH, H), jnp.float32),
            pltpu.VMEM((CH, H), jnp.float32),
            pltpu.VMEM((CH, H), jnp.float32),
            pltpu.VMEM((CH, H), jnp.float32),
            pltpu.VMEM((RPT, H), jnp.float32),
            pltpu.VMEM_SHARED((NP, H), jnp.float32),
            pltpu.SemaphoreType.DMA,
            pltpu.SemaphoreType.DMA,
            pltpu.SemaphoreType.DMA,
            pltpu.SemaphoreType.DMA,
        ],
    )
    def deg_kernel(dst_hbm, w_hbm, acc_hbm, dst_v, w_v, w0, w1, w2, w3,
                   zbuf_v, acc_sh, p0, p1, p2, p3):
        wwides = [w0, w1, w2, w3]
        ssems = [p0, p1, p2, p3]
        c = lax.axis_index("core")
        s = lax.axis_index("subcore")
        wid = c * NS + s
        pltpu.sync_copy(dst_hbm.at[wid], dst_v)
        pltpu.sync_copy(w_hbm.at[wid], w_v)
        _zero_shared_stripe(zbuf_v, acc_sh, s)
        plsc.subcore_barrier()

        @pl.loop(0, NJB)
        def _(j):
            for b in range(NBUF):
                @pl.when(j > 0)
                def _():
                    pltpu.make_async_copy(
                        wwides[b], acc_sh.at[dst_v.at[j, b]], ssems[b]).wait()
                for q in range(CH // 16):
                    wv = w_v[j, b, pl.ds(q * 16, 16)]
                    ridx = lax.iota(jnp.int32, 16) + q * 16
                    for f in range(H):
                        fidx = jnp.full((16,), f, dtype=jnp.int32)
                        plsc.store_scatter(wwides[b], [ridx, fidx], wv)
                pltpu.async_copy(wwides[b], acc_sh.at[dst_v.at[j, b]],
                                 ssems[b], add=True)

        for b in range(NBUF):
            pltpu.make_async_copy(wwides[b], acc_sh.at[dst_v.at[0, b]],
                                  ssems[b]).wait()
        plsc.subcore_barrier()
        pltpu.sync_copy(acc_sh.at[pl.ds(s * RPT, RPT)],
                        acc_hbm.at[c].at[pl.ds(s * RPT, RPT)])

    return deg_kernel(dst3, w3)


def _sc_spmm(hp, src3, dst3, w3):
    @functools.partial(
        pl.kernel,
        out_type=jax.ShapeDtypeStruct((NC, NP, H), jnp.float32),
        mesh=_mesh,
        compiler_params=_sc_params,
        scratch_types=[
            pltpu.VMEM((NJB, NBUF, CH), jnp.int32),
            pltpu.VMEM((NJB, NBUF, CH), jnp.int32),
            pltpu.VMEM((NJB, NBUF, CH), jnp.float32),
            pltpu.VMEM((CH, H), jnp.float32),
            pltpu.VMEM((CH, H), jnp.float32),
            pltpu.VMEM((CH, H), jnp.float32),
            pltpu.VMEM((CH, H), jnp.float32),
            pltpu.VMEM((CH, H), jnp.float32),
            pltpu.VMEM((CH, H), jnp.float32),
            pltpu.VMEM((CH, H), jnp.float32),
            pltpu.VMEM((CH, H), jnp.float32),
            pltpu.VMEM((RPT, H), jnp.float32),
            pltpu.VMEM_SHARED((NP, H), jnp.float32),
            pltpu.SemaphoreType.DMA,
            pltpu.SemaphoreType.DMA,
            pltpu.SemaphoreType.DMA,
            pltpu.SemaphoreType.DMA,
            pltpu.SemaphoreType.DMA,
            pltpu.SemaphoreType.DMA,
            pltpu.SemaphoreType.DMA,
            pltpu.SemaphoreType.DMA,
        ],
    )
    def spmm_kernel(hp_hbm, src_hbm, dst_hbm, w_hbm, acc_hbm,
                    src_v, dst_v, w_v, g0, g1, g2, g3, s0, s1, s2, s3,
                    zbuf_v, acc_sh, q0, q1, q2, q3, p0, p1, p2, p3):
        gbufs = [g0, g1, g2, g3]
        sbufs = [s0, s1, s2, s3]
        gsems = [q0, q1, q2, q3]
        ssems = [p0, p1, p2, p3]
        c = lax.axis_index("core")
        s = lax.axis_index("subcore")
        wid = c * NS + s
        pltpu.sync_copy(src_hbm.at[wid], src_v)
        pltpu.sync_copy(dst_hbm.at[wid], dst_v)
        pltpu.sync_copy(w_hbm.at[wid], w_v)
        _zero_shared_stripe(zbuf_v, acc_sh, s)
        plsc.subcore_barrier()

        @pl.loop(0, NJB)
        def _(j):
            for b in range(NBUF):
                pltpu.async_copy(hp_hbm.at[src_v.at[j, b]], gbufs[b],
                                 gsems[b])
            for b in range(NBUF):
                pltpu.make_async_copy(hp_hbm.at[src_v.at[j, b]], gbufs[b],
                                      gsems[b]).wait()

                @pl.when(j > 0)
                def _():
                    pltpu.make_async_copy(
                        sbufs[b], acc_sh.at[dst_v.at[j, b]], ssems[b]).wait()

                _scale_rows(gbufs[b], sbufs[b], w_v, j, b)
                pltpu.async_copy(sbufs[b], acc_sh.at[dst_v.at[j, b]],
                                 ssems[b], add=True)

        for b in range(NBUF):
            pltpu.make_async_copy(sbufs[b], acc_sh.at[dst_v.at[0, b]],
                                  ssems[b]).wait()
        plsc.subcore_barrier()
        pltpu.sync_copy(acc_sh.at[pl.ds(s * RPT, RPT)],
                        acc_hbm.at[c].at[pl.ds(s * RPT, RPT)])

    return spmm_kernel(hp, src3, dst3, w3)


def _tc_matmul1(x, W1):
    def body(x_ref, w_ref, h_ref):
        h_ref[...] = jnp.dot(x_ref[...], w_ref[...],
                             preferred_element_type=jnp.float32)
    return pl.pallas_call(
        body,
        out_shape=jax.ShapeDtypeStruct((N, H), jnp.float32),
    )(x, W1)


def _tc_prescale(h1, d0, d1):
    def body(h_ref, d0_ref, d1_ref, hp_ref, dinv_ref):
        deg = d0_ref[...] + d1_ref[...] + 1.0
        dinv = lax.rsqrt(deg)
        dinv_ref[...] = dinv
        hp_ref[...] = dinv * h_ref[...]
    return pl.pallas_call(
        body,
        out_shape=[jax.ShapeDtypeStruct((N, H), jnp.float32),
                   jax.ShapeDtypeStruct((N, H), jnp.float32)],
    )(h1, d0, d1)


def _tc_mid(a0, a1, hp1, dinv, W2, b1):
    def body(a0_ref, a1_ref, hp_ref, dinv_ref, w_ref, b_ref, hp2_ref):
        out1 = dinv_ref[...] * (a0_ref[...] + a1_ref[...] + hp_ref[...]) + b_ref[...]
        out1 = jnp.maximum(out1, 0.0)
        h2 = jnp.dot(out1, w_ref[...], preferred_element_type=jnp.float32)
        hp2_ref[...] = dinv_ref[...] * h2
    return pl.pallas_call(
        body,
        out_shape=jax.ShapeDtypeStruct((N, H), jnp.float32),
    )(a0, a1, hp1, dinv, W2, b1)


def _tc_head(a0, a1, hp2, dinv, b2, wlin_t, blin):
    def body(a0_ref, a1_ref, hp_ref, dinv_ref, b_ref, wl_ref, bl_ref, o_ref):
        out2 = dinv_ref[...] * (a0_ref[...] + a1_ref[...] + hp_ref[...]) + b_ref[...]
        out2 = jnp.maximum(out2, 0.0)
        g = jnp.sum(out2, axis=0, keepdims=True) * (1.0 / N)
        o_ref[...] = jnp.sum(g * wl_ref[...], axis=1, keepdims=True) + bl_ref[...]
    return pl.pallas_call(
        body,
        out_shape=jax.ShapeDtypeStruct((1, 1), jnp.float32),
    )(a0, a1, hp2, dinv, b2, wlin_t, blin)


@jax.jit
def kernel(x, edge_index, edge_attr, W1, b1, W2, b2, Wlin, blin):
    src = edge_index[0]
    dst = edge_index[1]
    pad = EPAD - E
    zi = jnp.zeros((pad,), dtype=jnp.int32)
    src3 = jnp.concatenate([src, zi]).reshape(NW, NJB, NBUF, CH)
    dst3 = jnp.concatenate([dst, zi]).reshape(NW, NJB, NBUF, CH)
    w3 = jnp.concatenate([edge_attr, jnp.zeros((pad,), jnp.float32)]).reshape(
        NW, NJB, NBUF, CH)

    h1 = _tc_matmul1(x, W1)
    dacc = _sc_degree(dst3, w3)
    hp1, dinv = _tc_prescale(h1, dacc[0, :N], dacc[1, :N])
    acc1 = _sc_spmm(hp1, src3, dst3, w3)
    hp2 = _tc_mid(acc1[0, :N], acc1[1, :N], hp1, dinv, W2, b1.reshape(1, H))
    acc2 = _sc_spmm(hp2, src3, dst3, w3)
    return _tc_head(acc2[0, :N], acc2[1, :N], hp2, dinv, b2.reshape(1, H),
                    Wlin.reshape(1, H), blin.reshape(1, 1))


# speedup vs baseline: 1.0011x; 1.0011x over previous
import functools

import jax
import jax.numpy as jnp
from jax import lax
from jax.experimental import pallas as pl
from jax.experimental.pallas import tpu as pltpu
from jax.experimental.pallas import tpu_sc as plsc

N = 10000
E = 320000
D = 128
H = 16

NC = 2
NS = 16
NW = NC * NS
CH = 128
EPW = -(-E // (NW * CH * 4)) * CH * 4
NJ = EPW // CH
EPAD = NW * EPW
RPT = 640
NP = NS * RPT
NBUF = 4
NJB = NJ // NBUF

_mesh = plsc.VectorSubcoreMesh(core_axis_name="core", subcore_axis_name="subcore")
_sc_params = pltpu.CompilerParams(needs_layout_passes=False,
                                  use_tc_tiling_on_sc=False)


def _scale_rows(gbuf, sbuf, w_v, j, b):
    for q in range(CH // 16):
        wv = w_v[j, b, pl.ds(q * 16, 16)]
        ridx = lax.iota(jnp.int32, 16) + q * 16
        for f in range(H):
            fidx = jnp.full((16,), f, dtype=jnp.int32)
            vals = plsc.load_gather(gbuf, [ridx, fidx])
            plsc.store_scatter(sbuf, [ridx, fidx], vals * wv)


def _zero_shared_stripe(zbuf_v, acc_sh, s):
    @pl.loop(0, RPT)
    def _(i):
        zbuf_v[i, :] = jnp.zeros((16,), jnp.float32)
    pltpu.sync_copy(zbuf_v, acc_sh.at[pl.ds(s * RPT, RPT)])


def _sc_degree(dst3, w3):
    @functools.partial(
        pl.kernel,
        out_type=jax.ShapeDtypeStruct((NC, NP, H), jnp.float32),
        mesh=_mesh,
        compiler_params=_sc_params,
        scratch_types=[
            pltpu.VMEM((NJB, NBUF, CH), jnp.int32),
            pltpu.VMEM((NJB, NBUF, CH), jnp.float32),
            pltpu.VMEM((CH, H), jnp.float32),
            pltpu.VMEM((CH, H), jnp.float32),
            pltpu.VMEM((CH, H), jnp.float32),
            pltpu.VMEM((CH, H), jnp.float32),
            pltpu.VMEM((RPT, H), jnp.float32),
            pltpu.VMEM_SHARED((NP, H), jnp.float32),
            pltpu.SemaphoreType.DMA,
            pltpu.SemaphoreType.DMA,
            pltpu.SemaphoreType.DMA,
            pltpu.SemaphoreType.DMA,
        ],
    )
    def deg_kernel(dst_hbm, w_hbm, acc_hbm, dst_v, w_v, w0, w1, w2, w3,
                   zbuf_v, acc_sh, p0, p1, p2, p3):
        wwides = [w0, w1, w2, w3]
        ssems = [p0, p1, p2, p3]
        c = lax.axis_index("core")
        s = lax.axis_index("subcore")
        wid = c * NS + s
        pltpu.sync_copy(dst_hbm.at[wid], dst_v)
        pltpu.sync_copy(w_hbm.at[wid], w_v)
        _zero_shared_stripe(zbuf_v, acc_sh, s)
        plsc.subcore_barrier()

        @pl.loop(0, NJB)
        def _(j):
            for b in range(NBUF):
                @pl.when(j > 0)
                def _():
                    pltpu.make_async_copy(
                        wwides[b], acc_sh.at[dst_v.at[j, b]], ssems[b]).wait()
                for q in range(CH // 16):
                    wv = w_v[j, b, pl.ds(q * 16, 16)]
                    ridx = lax.iota(jnp.int32, 16) + q * 16
                    for f in range(H):
                        fidx = jnp.full((16,), f, dtype=jnp.int32)
                        plsc.store_scatter(wwides[b], [ridx, fidx], wv)
                pltpu.async_copy(wwides[b], acc_sh.at[dst_v.at[j, b]],
                                 ssems[b], add=True)

        for b in range(NBUF):
            pltpu.make_async_copy(wwides[b], acc_sh.at[dst_v.at[0, b]],
                                  ssems[b]).wait()
        plsc.subcore_barrier()
        pltpu.sync_copy(acc_sh.at[pl.ds(s * RPT, RPT)],
                        acc_hbm.at[c].at[pl.ds(s * RPT, RPT)])

    return deg_kernel(dst3, w3)


def _sc_spmm(hp, src3, dst3, w3):
    @functools.partial(
        pl.kernel,
        out_type=jax.ShapeDtypeStruct((NC, NP, H), jnp.float32),
        mesh=_mesh,
        compiler_params=_sc_params,
        scratch_types=[
            pltpu.VMEM((NJB, NBUF, CH), jnp.int32),
            pltpu.VMEM((NJB, NBUF, CH), jnp.int32),
            pltpu.VMEM((NJB, NBUF, CH), jnp.float32),
            pltpu.VMEM((CH, H), jnp.float32),
            pltpu.VMEM((CH, H), jnp.float32),
            pltpu.VMEM((CH, H), jnp.float32),
            pltpu.VMEM((CH, H), jnp.float32),
            pltpu.VMEM((CH, H), jnp.float32),
            pltpu.VMEM((CH, H), jnp.float32),
            pltpu.VMEM((CH, H), jnp.float32),
            pltpu.VMEM((CH, H), jnp.float32),
            pltpu.VMEM((RPT, H), jnp.float32),
            pltpu.VMEM_SHARED((NP, H), jnp.float32),
            pltpu.SemaphoreType.DMA,
            pltpu.SemaphoreType.DMA,
            pltpu.SemaphoreType.DMA,
            pltpu.SemaphoreType.DMA,
            pltpu.SemaphoreType.DMA,
            pltpu.SemaphoreType.DMA,
            pltpu.SemaphoreType.DMA,
            pltpu.SemaphoreType.DMA,
        ],
    )
    def spmm_kernel(hp_hbm, src_hbm, dst_hbm, w_hbm, acc_hbm,
                    src_v, dst_v, w_v, g0, g1, g2, g3, s0, s1, s2, s3,
                    zbuf_v, acc_sh, q0, q1, q2, q3, p0, p1, p2, p3):
        gbufs = [g0, g1, g2, g3]
        sbufs = [s0, s1, s2, s3]
        gsems = [q0, q1, q2, q3]
        ssems = [p0, p1, p2, p3]
        c = lax.axis_index("core")
        s = lax.axis_index("subcore")
        wid = c * NS + s
        pltpu.sync_copy(src_hbm.at[wid], src_v)
        pltpu.sync_copy(dst_hbm.at[wid], dst_v)
        pltpu.sync_copy(w_hbm.at[wid], w_v)
        _zero_shared_stripe(zbuf_v, acc_sh, s)
        plsc.subcore_barrier()

        @pl.loop(0, NJB)
        def _(j):
            for b in range(NBUF):
                pltpu.async_copy(hp_hbm.at[src_v.at[j, b]], gbufs[b],
                                 gsems[b])
            for b in range(NBUF):
                pltpu.make_async_copy(hp_hbm.at[src_v.at[j, b]], gbufs[b],
                                      gsems[b]).wait()

                @pl.when(j > 0)
                def _():
                    pltpu.make_async_copy(
                        sbufs[b], acc_sh.at[dst_v.at[j, b]], ssems[b]).wait()

                _scale_rows(gbufs[b], sbufs[b], w_v, j, b)
                pltpu.async_copy(sbufs[b], acc_sh.at[dst_v.at[j, b]],
                                 ssems[b], add=True)

        for b in range(NBUF):
            pltpu.make_async_copy(sbufs[b], acc_sh.at[dst_v.at[0, b]],
                                  ssems[b]).wait()
        plsc.subcore_barrier()
        pltpu.sync_copy(acc_sh.at[pl.ds(s * RPT, RPT)],
                        acc_hbm.at[c].at[pl.ds(s * RPT, RPT)])

    return spmm_kernel(hp, src3, dst3, w3)


def _tc_prescale(x, W1, d0, d1):
    def body(x_ref, w_ref, d0_ref, d1_ref, hp_ref, dinv_ref):
        deg = d0_ref[...] + d1_ref[...] + 1.0
        dinv = lax.rsqrt(deg)
        h = jnp.dot(x_ref[...], w_ref[...], preferred_element_type=jnp.float32)
        dinv_ref[...] = dinv
        hp_ref[...] = dinv * h
    return pl.pallas_call(
        body,
        out_shape=[jax.ShapeDtypeStruct((N, H), jnp.float32),
                   jax.ShapeDtypeStruct((N, H), jnp.float32)],
    )(x, W1, d0, d1)


def _tc_mid(a0, a1, hp1, dinv, W2, b1):
    def body(a0_ref, a1_ref, hp_ref, dinv_ref, w_ref, b_ref, hp2_ref):
        out1 = dinv_ref[...] * (a0_ref[...] + a1_ref[...] + hp_ref[...]) + b_ref[...]
        out1 = jnp.maximum(out1, 0.0)
        h2 = jnp.dot(out1, w_ref[...], preferred_element_type=jnp.float32)
        hp2_ref[...] = dinv_ref[...] * h2
    return pl.pallas_call(
        body,
        out_shape=jax.ShapeDtypeStruct((N, H), jnp.float32),
    )(a0, a1, hp1, dinv, W2, b1)


def _tc_head(a0, a1, hp2, dinv, b2, wlin_t, blin):
    def body(a0_ref, a1_ref, hp_ref, dinv_ref, b_ref, wl_ref, bl_ref, o_ref):
        out2 = dinv_ref[...] * (a0_ref[...] + a1_ref[...] + hp_ref[...]) + b_ref[...]
        out2 = jnp.maximum(out2, 0.0)
        g = jnp.sum(out2, axis=0, keepdims=True) * (1.0 / N)
        o_ref[...] = jnp.sum(g * wl_ref[...], axis=1, keepdims=True) + bl_ref[...]
    return pl.pallas_call(
        body,
        out_shape=jax.ShapeDtypeStruct((1, 1), jnp.float32),
    )(a0, a1, hp2, dinv, b2, wlin_t, blin)


@jax.jit
def kernel(x, edge_index, edge_attr, W1, b1, W2, b2, Wlin, blin):
    src = edge_index[0]
    dst = edge_index[1]
    pad = EPAD - E
    zi = jnp.zeros((pad,), dtype=jnp.int32)
    src3 = jnp.concatenate([src, zi]).reshape(NW, NJB, NBUF, CH)
    dst3 = jnp.concatenate([dst, zi]).reshape(NW, NJB, NBUF, CH)
    w3 = jnp.concatenate([edge_attr, jnp.zeros((pad,), jnp.float32)]).reshape(
        NW, NJB, NBUF, CH)

    dacc = _sc_degree(dst3, w3)
    hp1, dinv = _tc_prescale(x, W1, dacc[0, :N], dacc[1, :N])
    acc1 = _sc_spmm(hp1, src3, dst3, w3)
    hp2 = _tc_mid(acc1[0, :N], acc1[1, :N], hp1, dinv, W2, b1.reshape(1, H))
    acc2 = _sc_spmm(hp2, src3, dst3, w3)
    return _tc_head(acc2[0, :N], acc2[1, :N], hp2, dinv, b2.reshape(1, H),
                    Wlin.reshape(1, H), blin.reshape(1, 1))
